# Initial kernel scaffold; baseline (speedup 1.0000x reference)
import functools

import jax
import jax.numpy as jnp
from jax import lax
from jax.experimental import pallas as pl
from jax.experimental.pallas import tpu as pltpu
from jax.experimental.pallas import tpu_sc as plsc

EPS = 1e-5

V = 60000
P = 20
IN_DIM = 4
HID = 16
OUT = 16
B, GH, GW, GZ = 2, 256, 256, 16
NCELL = B * GH * GW * GZ

BV = 480
NBLK = V // BV
PH = P * HID

NW = 32
ROWS_PER_W = NCELL // NW
WIN = 4000
NWIN = V // WIN
VECS = WIN // 16
CAP = 4224
ZROWS = 1024
NFILL = ROWS_PER_W // ZROWS
GCH = 8


def _mlp_body(f_ref, np_ref, c_ref, w1p_ref, b1t_ref, g1t_ref, be1t_ref,
              t_ref, t2_ref, s_ref, w2_ref, b2_ref, g2_ref, be2_ref,
              x_ref, lin_ref):
    hi = jax.lax.Precision.HIGHEST
    f = f_ref[...]
    x1 = jnp.dot(f, w1p_ref[...], precision=hi) + b1t_ref[...]
    mu_g = jnp.dot(x1, t_ref[...], precision=hi)
    mu = jnp.dot(mu_g, t2_ref[...], precision=hi)
    xc = x1 - mu
    var_g = jnp.dot(xc * xc, t_ref[...], precision=hi)
    var = jnp.dot(var_g, t2_ref[...], precision=hi)
    xn = xc * lax.rsqrt(var + EPS) * g1t_ref[...] + be1t_ref[...]
    xr = jnp.maximum(xn, 0.0)
    npts = np_ref[...]
    lane_p = lax.broadcasted_iota(jnp.int32, (BV, PH), 1) // HID
    xm = jnp.where(lane_p < npts, xr, 0.0)
    pooled_pre = jnp.dot(xm, s_ref[...], precision=hi)
    pooled = (jnp.dot(pooled_pre, w2_ref[...], precision=hi)
              + b2_ref[...] * npts.astype(jnp.float32))
    mu2 = jnp.mean(pooled, axis=1, keepdims=True)
    xc2 = pooled - mu2
    var2 = jnp.mean(xc2 * xc2, axis=1, keepdims=True)
    x_ref[...] = xc2 * lax.rsqrt(var2 + EPS) * g2_ref[...] + be2_ref[...]
    c = c_ref[...]
    lin_ref[...] = (((c[:, 0:1] * GH + c[:, 1:2]) * GW + c[:, 2:3]) * GZ
                    + c[:, 3:4])


def _tc_mlp(feats2d, npts2d, coords, w1p, b1t, g1t, be1t, t, t2, s, w2, b2,
            g2, be2, interpret=False):
    bcast = lambda shape: pl.BlockSpec(shape, lambda i: (0,) * len(shape))
    return pl.pallas_call(
        _mlp_body,
        grid=(NBLK,),
        in_specs=[
            pl.BlockSpec((BV, P * IN_DIM), lambda i: (i, 0)),
            pl.BlockSpec((BV, 1), lambda i: (i, 0)),
            pl.BlockSpec((BV, 4), lambda i: (i, 0)),
            bcast((P * IN_DIM, PH)),
            bcast((1, PH)),
            bcast((1, PH)),
            bcast((1, PH)),
            bcast((PH, P)),
            bcast((P, PH)),
            bcast((PH, HID)),
            bcast((HID, OUT)),
            bcast((1, OUT)),
            bcast((1, OUT)),
            bcast((1, OUT)),
        ],
        out_specs=[
            pl.BlockSpec((BV, OUT), lambda i: (i, 0)),
            pl.BlockSpec((BV, 1), lambda i: (i, 0)),
        ],
        out_shape=[
            jax.ShapeDtypeStruct((V, OUT), jnp.float32),
            jax.ShapeDtypeStruct((V, 1), jnp.int32),
        ],
        interpret=interpret,
    )(feats2d, npts2d, coords, w1p, b1t, g1t, be1t, t, t2, s, w2, b2, g2, be2)


def _sc_body(x_hbm, lin_hbm, dense_hbm,
             visited, linwin, klin, kv, rows, zbuf, scat_idx, tail,
             sem_fill, sem_g, sem_s):
    wid = lax.axis_index("s") * 2 + lax.axis_index("c")
    row0 = wid * ROWS_PER_W

    def _z(i, _):
        zbuf[i, :] = jnp.zeros((16,), jnp.float32)
        return 0
    lax.fori_loop(0, ZROWS, _z, 0)
    for k in range(NFILL):
        pltpu.async_copy(
            zbuf, dense_hbm.at[pl.ds(row0 + k * ZROWS, ZROWS)], sem_fill)

    def _zv(i, _):
        visited[pl.ds(i * 16, 16)] = jnp.zeros((16,), jnp.int32)
        return 0
    lax.fori_loop(0, ROWS_PER_W // 16, _zv, 0)

    one = jnp.ones((16,), jnp.int32)

    def _vec(j, off, wbase):
        i = VECS - 1 - j
        lv = linwin[pl.ds(i * 16, 16)]
        mine = (lv >> 16) == wid
        lid = lv & 0xFFFF
        seen = plsc.load_gather(visited, [lid])
        _, lastocc = plsc.scan_count(lv)
        keep = mine & lastocc & (seen == 0)
        plsc.store_scatter(visited, [lid], one, mask=keep)
        vvec = wbase + lax.iota(jnp.int32, 16) + i * 16
        plsc.store_compressed(klin.at[pl.ds(off, 16)], lv, mask=keep)
        plsc.store_compressed(kv.at[pl.ds(off, 16)], vvec, mask=keep)
        cnt = plsc.all_reduce_population_count(keep)
        return off + cnt[0]

    def _win(t, off):
        w = NWIN - 1 - t
        pltpu.sync_copy(lin_hbm.at[pl.ds(w * WIN, WIN)], linwin)
        return lax.fori_loop(0, VECS, lambda j, o: _vec(j, o, w * WIN), off)

    n = lax.fori_loop(0, NWIN, _win, jnp.int32(0))

    @pl.when(n > 0)
    def _flush():
        pad_lin = jnp.full((16,), klin[0], jnp.int32)
        pad_v = jnp.full((16,), kv[0], jnp.int32)
        for i in range(8):
            klin[pl.ds(n + i * 16, 16)] = pad_lin
            kv[pl.ds(n + i * 16, 16)] = pad_v

        nch = (n + 127) // 128
        for k in range(NFILL):
            pltpu.make_async_copy(
                zbuf, dense_hbm.at[pl.ds(row0 + k * ZROWS, ZROWS)],
                sem_fill).wait()

        def _grp(g, _):
            base = g * GCH
            nj = jnp.minimum(nch - base, GCH)

            def _gather(j, _):
                c0 = (base + j) * 128
                pltpu.sync_copy(klin.at[pl.ds(c0, 128)], scat_idx.at[j])
                pltpu.async_copy(x_hbm.at[kv.at[pl.ds(c0, 128)]],
                                 rows.at[pl.ds(j * 128, 128)], sem_g)
                return 0
            lax.fori_loop(0, nj, _gather, 0)

            def _gwait(j, _):
                pltpu.make_async_copy(
                    x_hbm.at[kv.at[pl.ds(0, 128)]],
                    rows.at[pl.ds(0, 128)], sem_g).wait()
                return 0
            lax.fori_loop(0, nj, _gwait, 0)

            def _scat(j, _):
                pltpu.async_copy(rows.at[pl.ds(j * 128, 128)],
                                 dense_hbm.at[scat_idx.at[j]], sem_s)
                return 0
            lax.fori_loop(0, nj, _scat, 0)

            def _swait(j, _):
                pltpu.make_async_copy(rows.at[pl.ds(0, 128)],
                                      dense_hbm.at[scat_idx.at[0]],
                                      sem_s).wait()
                return 0
            lax.fori_loop(0, nj, _swait, 0)
            return 0
        lax.fori_loop(0, (nch + GCH - 1) // GCH, _grp, 0)

    @pl.when(n == 0)
    def _drain_fill():
        for k in range(NFILL):
            pltpu.make_async_copy(
                zbuf, dense_hbm.at[pl.ds(row0 + k * ZROWS, ZROWS)],
                sem_fill).wait()


def _sc_scatter(x, lin, interpret=False):
    mesh = plsc.VectorSubcoreMesh(core_axis_name="c", subcore_axis_name="s")
    f = pl.kernel(
        _sc_body,
        out_type=jax.ShapeDtypeStruct((NCELL, OUT), jnp.float32),
        mesh=mesh,
        scratch_types=[
            pltpu.VMEM((ROWS_PER_W,), jnp.int32),
            pltpu.VMEM((WIN,), jnp.int32),
            pltpu.VMEM((CAP,), jnp.int32),
            pltpu.VMEM((CAP,), jnp.int32),
            pltpu.VMEM((GCH * 128, OUT), jnp.float32),
            pltpu.VMEM((ZROWS, OUT), jnp.float32),
            pltpu.VMEM((GCH, 128), jnp.int32),
            pltpu.VMEM((16,), jnp.int32),
            pltpu.SemaphoreType.DMA,
            pltpu.SemaphoreType.DMA,
            pltpu.SemaphoreType.DMA,
        ],
        interpret=interpret,
    )
    return f(x, lin)


def kernel(features, num_points, coords, batch_size, grid_h, grid_w, grid_z,
           W1, b1, g1, be1, W2, b2, g2, be2):
    del batch_size, grid_h, grid_w, grid_z
    feats2d = features.reshape(V, P * IN_DIM)
    npts2d = num_points.reshape(V, 1)

    eye_p = jnp.eye(P, dtype=jnp.float32)
    w1p = jnp.einsum("pq,ih->piqh", eye_p, W1).reshape(P * IN_DIM, PH)
    tile = lambda v: jnp.tile(v, P).reshape(1, PH)
    b1t, g1t, be1t = tile(b1), tile(g1), tile(be1)
    t = jnp.repeat(jnp.eye(P, dtype=jnp.float32), HID, axis=0) / HID
    t2 = jnp.repeat(jnp.eye(P, dtype=jnp.float32), HID, axis=1)
    s = jnp.tile(jnp.eye(HID, dtype=jnp.float32), (P, 1))

    x, lin = _tc_mlp(feats2d, npts2d, coords, w1p, b1t, g1t, be1t, t, t2, s,
                     W2, b2.reshape(1, OUT), g2.reshape(1, OUT),
                     be2.reshape(1, OUT))
    dense = _sc_scatter(x, lin.reshape(V))
    return dense.reshape(B, GH, GW, GZ, OUT)


# speedup vs baseline: 4.6268x; 4.6268x over previous
import functools

import jax
import jax.numpy as jnp
from jax import lax
from jax.experimental import pallas as pl
from jax.experimental.pallas import tpu as pltpu
from jax.experimental.pallas import tpu_sc as plsc

EPS = 1e-5

V = 60000
P = 20
IN_DIM = 4
HID = 16
OUT = 16
B, GH, GW, GZ = 2, 256, 256, 16
NCELL = B * GH * GW * GZ

BV = 480
NBLK = V // BV
PH = P * HID

NW = 32
ROWS_PER_W = NCELL // NW
WIN = 4000
NWIN = V // WIN
VECS = WIN // 16
CAP = 4224
ZROWS = 1024
NFILL = ROWS_PER_W // ZROWS
GCH = 8


def _mlp_body(f_ref, np_ref, c_ref, w1p_ref, b1t_ref, be1t_ref,
              t_ref, t2_ref, t2g_ref, s_ref, w2_ref, b2_ref, g2_ref, be2_ref,
              lanep_ref, x_ref, lin_ref):
    hi = jax.lax.Precision.DEFAULT
    f = f_ref[...]
    x1 = jnp.dot(f, w1p_ref[...], precision=hi) + b1t_ref[...]
    mu_g = jnp.dot(x1, t_ref[...], precision=hi)
    mu = jnp.dot(mu_g, t2_ref[...], precision=hi)
    xc = x1 - mu
    var_g = jnp.dot(xc * xc, t_ref[...], precision=hi)
    scale_g = lax.rsqrt(var_g + EPS)
    scale = jnp.dot(scale_g, t2g_ref[...], precision=hi)
    xn = xc * scale + be1t_ref[...]
    xr = jnp.maximum(xn, 0.0)
    npts = np_ref[...]
    xm = jnp.where(lanep_ref[...] < npts, xr, 0.0)
    pooled_pre = jnp.dot(xm, s_ref[...], precision=hi)
    pooled = (jnp.dot(pooled_pre, w2_ref[...], precision=hi)
              + b2_ref[...] * npts.astype(jnp.float32))
    mu2 = jnp.mean(pooled, axis=1, keepdims=True)
    xc2 = pooled - mu2
    var2 = jnp.mean(xc2 * xc2, axis=1, keepdims=True)
    x_ref[...] = xc2 * lax.rsqrt(var2 + EPS) * g2_ref[...] + be2_ref[...]
    c = c_ref[...]
    lin_ref[...] = (((c[:, 0:1] * GH + c[:, 1:2]) * GW + c[:, 2:3]) * GZ
                    + c[:, 3:4])


def _tc_mlp(feats2d, npts2d, coords, w1p, b1t, be1t, t, t2, t2g, s, w2, b2,
            g2, be2, lanep, interpret=False):
    bcast = lambda shape: pl.BlockSpec(shape, lambda i: (0,) * len(shape))
    return pl.pallas_call(
        _mlp_body,
        grid=(NBLK,),
        in_specs=[
            pl.BlockSpec((BV, P * IN_DIM), lambda i: (i, 0)),
            pl.BlockSpec((BV, 1), lambda i: (i, 0)),
            pl.BlockSpec((BV, 4), lambda i: (i, 0)),
            bcast((P * IN_DIM, PH)),
            bcast((1, PH)),
            bcast((1, PH)),
            bcast((PH, P)),
            bcast((P, PH)),
            bcast((P, PH)),
            bcast((PH, HID)),
            bcast((HID, OUT)),
            bcast((1, OUT)),
            bcast((1, OUT)),
            bcast((1, OUT)),
            bcast((1, PH)),
        ],
        out_specs=[
            pl.BlockSpec((BV, OUT), lambda i: (i, 0)),
            pl.BlockSpec((BV, 1), lambda i: (i, 0)),
        ],
        out_shape=[
            jax.ShapeDtypeStruct((V, OUT), jnp.float32),
            jax.ShapeDtypeStruct((V, 1), jnp.int32),
        ],
        interpret=interpret,
    )(feats2d, npts2d, coords, w1p, b1t, be1t, t, t2, t2g, s, w2, b2, g2, be2,
      lanep)


def _sc_body(x_hbm, lin_hbm, dense_hbm,
             visited, linwin, klin, kv, rows, zbuf, scat_idx, tail,
             sem_fill, sem_g, sem_s):
    wid = lax.axis_index("s") * 2 + lax.axis_index("c")
    row0 = wid * ROWS_PER_W

    def _z(i, _):
        zbuf[i, :] = jnp.zeros((16,), jnp.float32)
        return 0
    lax.fori_loop(0, ZROWS, _z, 0)
    for k in range(NFILL):
        pltpu.async_copy(
            zbuf, dense_hbm.at[pl.ds(row0 + k * ZROWS, ZROWS)], sem_fill)

    def _zv(i, _):
        visited[pl.ds(i * 16, 16)] = jnp.zeros((16,), jnp.int32)
        return 0
    lax.fori_loop(0, ROWS_PER_W // 16, _zv, 0)

    one = jnp.ones((16,), jnp.int32)

    def _vec(j, off, wbase):
        i = VECS - 1 - j
        lv = linwin[pl.ds(i * 16, 16)]
        mine = (lv >> 16) == wid
        lid = lv & 0xFFFF
        seen = plsc.load_gather(visited, [lid])
        _, lastocc = plsc.scan_count(lv)
        keep = mine & lastocc & (seen == 0)
        plsc.store_scatter(visited, [lid], one, mask=keep)
        vvec = wbase + lax.iota(jnp.int32, 16) + i * 16
        plsc.store_compressed(klin.at[pl.ds(off, 16)], lv, mask=keep)
        plsc.store_compressed(kv.at[pl.ds(off, 16)], vvec, mask=keep)
        cnt = plsc.all_reduce_population_count(keep)
        return off + cnt[0]

    def _win(t, off):
        w = NWIN - 1 - t
        pltpu.sync_copy(lin_hbm.at[pl.ds(w * WIN, WIN)], linwin)
        return lax.fori_loop(0, VECS, lambda j, o: _vec(j, o, w * WIN), off)

    n = lax.fori_loop(0, NWIN, _win, jnp.int32(0))

    @pl.when(n > 0)
    def _flush():
        pad_lin = jnp.full((16,), klin[pl.ds(0, 16)][0], jnp.int32)
        pad_v = jnp.full((16,), kv[pl.ds(0, 16)][0], jnp.int32)
        for i in range(8):
            klin[pl.ds(n + i * 16, 16)] = pad_lin
            kv[pl.ds(n + i * 16, 16)] = pad_v

        nch = (n + 127) // 128
        for k in range(NFILL):
            pltpu.make_async_copy(
                zbuf, dense_hbm.at[pl.ds(row0 + k * ZROWS, ZROWS)],
                sem_fill).wait()

        def _grp(g, _):
            base = g * GCH
            nj = jnp.minimum(nch - base, GCH)

            def _gather(j, _):
                c0 = (base + j) * 128
                for m in range(8):
                    scat_idx[j, pl.ds(m * 16, 16)] = klin[
                        pl.ds(c0 + m * 16, 16)]
                pltpu.async_copy(x_hbm.at[kv.at[pl.ds(c0, 128)]],
                                 rows.at[pl.ds(j * 128, 128)], sem_g)
                return 0
            lax.fori_loop(0, nj, _gather, 0)

            def _gwait(j, _):
                pltpu.make_async_copy(
                    x_hbm.at[kv.at[pl.ds(0, 128)]],
                    rows.at[pl.ds(0, 128)], sem_g).wait()
                return 0
            lax.fori_loop(0, nj, _gwait, 0)

            def _scat(j, _):
                pltpu.async_copy(rows.at[pl.ds(j * 128, 128)],
                                 dense_hbm.at[scat_idx.at[j]], sem_s)
                return 0
            lax.fori_loop(0, nj, _scat, 0)

            def _swait(j, _):
                pltpu.make_async_copy(rows.at[pl.ds(0, 128)],
                                      dense_hbm.at[scat_idx.at[0]],
                                      sem_s).wait()
                return 0
            lax.fori_loop(0, nj, _swait, 0)
            return 0
        lax.fori_loop(0, (nch + GCH - 1) // GCH, _grp, 0)

    @pl.when(n == 0)
    def _drain_fill():
        for k in range(NFILL):
            pltpu.make_async_copy(
                zbuf, dense_hbm.at[pl.ds(row0 + k * ZROWS, ZROWS)],
                sem_fill).wait()


def _sc_scatter(x, lin, interpret=False):
    mesh = plsc.VectorSubcoreMesh(core_axis_name="c", subcore_axis_name="s")
    f = pl.kernel(
        _sc_body,
        out_type=jax.ShapeDtypeStruct((NCELL, OUT), jnp.float32),
        mesh=mesh,
        scratch_types=[
            pltpu.VMEM((ROWS_PER_W,), jnp.int32),
            pltpu.VMEM((WIN,), jnp.int32),
            pltpu.VMEM((CAP,), jnp.int32),
            pltpu.VMEM((CAP,), jnp.int32),
            pltpu.VMEM((GCH * 128, OUT), jnp.float32),
            pltpu.VMEM((ZROWS, OUT), jnp.float32),
            pltpu.VMEM((GCH, 128), jnp.int32),
            pltpu.VMEM((16,), jnp.int32),
            pltpu.SemaphoreType.DMA,
            pltpu.SemaphoreType.DMA,
            pltpu.SemaphoreType.DMA,
        ],
        compiler_params=pltpu.CompilerParams(
            needs_layout_passes=False, use_tc_tiling_on_sc=False),
        interpret=interpret,
    )
    return f(x, lin)


def kernel(features, num_points, coords, batch_size, grid_h, grid_w, grid_z,
           W1, b1, g1, be1, W2, b2, g2, be2):
    del batch_size, grid_h, grid_w, grid_z
    feats2d = features.reshape(V, P * IN_DIM)
    npts2d = num_points.reshape(V, 1)

    eye_p = jnp.eye(P, dtype=jnp.float32)
    w1p = jnp.einsum("pq,ih->piqh", eye_p, W1).reshape(P * IN_DIM, PH)
    tile = lambda v: jnp.tile(v, P).reshape(1, PH)
    b1t, be1t = tile(b1), tile(be1)
    t = jnp.repeat(jnp.eye(P, dtype=jnp.float32), HID, axis=0) / HID
    t2 = jnp.repeat(jnp.eye(P, dtype=jnp.float32), HID, axis=1)
    t2g = t2 * jnp.tile(g1, P)[None, :]
    s = jnp.tile(jnp.eye(HID, dtype=jnp.float32), (P, 1))
    lanep = (jnp.arange(PH, dtype=jnp.int32) // HID).reshape(1, PH)

    x, lin = _tc_mlp(feats2d, npts2d, coords, w1p, b1t, be1t, t, t2, t2g, s,
                     W2, b2.reshape(1, OUT), g2.reshape(1, OUT),
                     be2.reshape(1, OUT), lanep)
    dense = _sc_scatter(x, lin.reshape(V))
    return dense.reshape(B, GH, GW, GZ, OUT)
